# Initial kernel scaffold; baseline (speedup 1.0000x reference)
import jax
import jax.numpy as jnp
from jax.experimental import pallas as pl


def kernel(x, edge_index, edge_weight, W_gcn, b_gcn, gamma, beta, W1, b1, W2, b2):
    raise NotImplementedError("write your pallas kernel here")


# speedup vs baseline: 18.5121x; 18.5121x over previous
import functools

import jax
import jax.numpy as jnp
from jax import lax
from jax.experimental import pallas as pl
from jax.experimental.pallas import tpu as pltpu
from jax.experimental.pallas import tpu_sc as plsc

N = 10000
E = 320000
D_IN = 128
H1 = 64

NC = 2
NS = 16
NW = NC * NS

C = 128
EW = 10240
NCHUNK = EW // C
E_PAD = NW * EW
N_PAD = 10240
NSL = N_PAD // NS


def _mm_body(x_ref, w_ref, o_ref):
    o_ref[...] = lax.dot_general(
        x_ref[...], w_ref[...], (((1,), (1,)), ((), ())),
        preferred_element_type=jnp.float32)


def _epi_body(agg_ref, b_ref, g_ref, be_ref, w1_ref, b1_ref, w2_ref, b2_ref,
              o_ref):
    agg = agg_ref[0] + agg_ref[1]
    out = jnp.maximum(agg + b_ref[...], 0.0)
    mu = jnp.mean(out, axis=-1, keepdims=True)
    var = jnp.mean((out - mu) * (out - mu), axis=-1, keepdims=True)
    ln = (out - mu) * lax.rsqrt(var + 1e-5) * g_ref[...] + be_ref[...]
    pooled = jnp.mean(ln, axis=0, keepdims=True)
    z = lax.dot_general(pooled, w1_ref[...], (((1,), (1,)), ((), ())),
                        preferred_element_type=jnp.float32) + b1_ref[...]
    z = jnp.maximum(z, 0.0)
    o_ref[...] = lax.dot_general(z, w2_ref[...], (((1,), (1,)), ((), ())),
                                 preferred_element_type=jnp.float32) + b2_ref[...]


def _sc_body(srcp, dstp, ewp, h_hbm, agg_out,
             src_v, dst_v, ew_v, norm_v, deg_v, degsum_v, dinv_sl, rows_v,
             norm_c, degs_s, dinv_s, agg_s, sem_g0, sem_g1, sem_m):
    c = lax.axis_index("c")
    s = lax.axis_index("s")
    w = s * NC + c
    t = s

    zero16 = jnp.zeros((16,), jnp.float32)

    def zbody(i, _):
        deg_v[pl.ds(i * 16, 16)] = zero16
        return ()
    lax.fori_loop(0, N_PAD // 16, zbody, ())

    for k in range(2):
        wk = t * 2 + k
        pltpu.sync_copy(dstp.at[wk], dst_v)
        pltpu.sync_copy(ewp.at[wk], ew_v)

        def dchunk(ci, _):
            for j in range(C // 16):
                sl = pl.ds(j * 16, 16)
                plsc.addupdate_scatter(deg_v, [dst_v[ci, sl]], ew_v[ci, sl])
            return ()
        lax.fori_loop(0, NCHUNK, dchunk, ())

    pltpu.sync_copy(deg_v, degs_s.at[t])
    plsc.subcore_barrier()

    descs = []
    for p in range(NS):
        descs.append(pltpu.async_copy(
            degs_s.at[p, pl.ds(t * NSL, NSL)], degsum_v.at[p], sem_m))
    for d in descs:
        d.wait()

    def nbody(i, _):
        sl = pl.ds(i * 16, 16)
        acc = degsum_v[0, sl]
        for p in range(1, NS):
            acc = acc + degsum_v[p, sl]
        dg = acc + 1.0
        ii = plsc.bitcast(dg, jnp.int32)
        ii = jnp.int32(0x5F3759DF) - lax.shift_right_arithmetic(ii, 1)
        y = plsc.bitcast(ii, jnp.float32)
        for _ in range(4):
            y = y * (1.5 - 0.5 * dg * y * y)
        dinv_sl[sl] = y
        return ()
    lax.fori_loop(0, NSL // 16, nbody, ())

    pltpu.sync_copy(dinv_sl, dinv_s.at[pl.ds(t * NSL, NSL)])
    plsc.subcore_barrier()
    pltpu.sync_copy(dinv_s, deg_v)

    for q in range(NSL // C):
        base = t * NSL + q * C
        pltpu.sync_copy(h_hbm.at[pl.ds(base, C)], rows_v.at[0])

        def sbody(g, _):
            dv = deg_v[pl.ds(base + g * 16, 16)]
            dv2 = dv * dv * 0.5
            for k in range(16):
                bv = jnp.full((16,), dv2[k], jnp.float32)
                e = g * 16 + k
                for j in range(H1 // 16):
                    sl = pl.ds(j * 16, 16)
                    rows_v[0, e, sl] = rows_v[0, e, sl] * bv
            return ()
        lax.fori_loop(0, C // 16, sbody, ())
        pltpu.sync_copy(rows_v.at[0], agg_s.at[pl.ds(base, C)])

    pltpu.sync_copy(srcp.at[w], src_v)
    pltpu.sync_copy(dstp.at[w], dst_v)
    pltpu.sync_copy(ewp.at[w], ew_v)

    def nrm(ci, _):
        for j in range(C // 16):
            sl = pl.ds(j * 16, 16)
            gs = plsc.load_gather(deg_v, [src_v[ci, sl]])
            gd = plsc.load_gather(deg_v, [dst_v[ci, sl]])
            norm_v[ci, sl] = gs * ew_v[ci, sl] * gd
        return ()
    lax.fori_loop(0, NCHUNK, nrm, ())

    plsc.subcore_barrier()

    pltpu.async_copy(h_hbm.at[src_v.at[0]], rows_v.at[0], sem_g0)
    pltpu.async_copy(h_hbm.at[src_v.at[1]], rows_v.at[1], sem_g1)

    def mainloop(step, _):
        for b in range(2):
            ci = step * 2 + b
            sem = sem_g0 if b == 0 else sem_g1
            pltpu.make_async_copy(
                h_hbm.at[src_v.at[ci]], rows_v.at[b], sem).wait()

            for g in range(C // 16):
                norm_c[pl.ds(g * 16, 16)] = norm_v[ci, pl.ds(g * 16, 16)]

            def scale(g, _):
                nv = norm_c[pl.ds(g * 16, 16)]
                for k in range(16):
                    bv = jnp.full((16,), nv[k], jnp.float32)
                    e = g * 16 + k
                    for j in range(H1 // 16):
                        sl = pl.ds(j * 16, 16)
                        rows_v[b, e, sl] = rows_v[b, e, sl] * bv
                return ()
            lax.fori_loop(0, C // 16, scale, ())

            pltpu.sync_copy(rows_v.at[b], agg_s.at[dst_v.at[ci]], add=True)

            @pl.when(ci + 2 < NCHUNK)
            def _():
                pltpu.async_copy(h_hbm.at[src_v.at[ci + 2]], rows_v.at[b], sem)
        return ()
    lax.fori_loop(0, NCHUNK // 2, mainloop, ())

    plsc.subcore_barrier()
    pltpu.sync_copy(agg_s.at[pl.ds(t * NSL, NSL)],
                    agg_out.at[c, pl.ds(t * NSL, NSL)])


def kernel(x, edge_index, edge_weight, W_gcn, b_gcn, gamma, beta, W1, b1, W2,
           b2):
    src = edge_index[0]
    dst = edge_index[1]
    pad = E_PAD - E
    srcp = jnp.concatenate([src, jnp.zeros((pad,), src.dtype)])
    dstp = jnp.concatenate([dst, jnp.zeros((pad,), dst.dtype)])
    ewp = jnp.concatenate([edge_weight, jnp.zeros((pad,), edge_weight.dtype)])
    srcp = srcp.reshape(NW, NCHUNK, C)
    dstp = dstp.reshape(NW, NCHUNK, C)
    ewp = ewp.reshape(NW, NCHUNK, C)

    x_pad = jnp.concatenate(
        [x, jnp.zeros((N_PAD - N, D_IN), x.dtype)], axis=0)

    h = pl.pallas_call(
        _mm_body,
        out_shape=jax.ShapeDtypeStruct((N_PAD, H1), jnp.float32),
    )(x_pad, W_gcn)

    mesh = plsc.VectorSubcoreMesh(
        core_axis_name="c", subcore_axis_name="s",
        num_cores=NC, num_subcores=NS)

    sc = functools.partial(
        pl.kernel,
        out_type=jax.ShapeDtypeStruct((NC, N_PAD, H1), jnp.float32),
        mesh=mesh,
        compiler_params=pltpu.CompilerParams(
            needs_layout_passes=False, use_tc_tiling_on_sc=False),
        scratch_types=[
            pltpu.VMEM((NCHUNK, C), jnp.int32),
            pltpu.VMEM((NCHUNK, C), jnp.int32),
            pltpu.VMEM((NCHUNK, C), jnp.float32),
            pltpu.VMEM((NCHUNK, C), jnp.float32),
            pltpu.VMEM((N_PAD,), jnp.float32),
            pltpu.VMEM((NS, NSL), jnp.float32),
            pltpu.VMEM((NSL,), jnp.float32),
            pltpu.VMEM((2, C, H1), jnp.float32),
            pltpu.VMEM((C,), jnp.float32),
            pltpu.VMEM_SHARED((NS, N_PAD), jnp.float32),
            pltpu.VMEM_SHARED((N_PAD,), jnp.float32),
            pltpu.VMEM_SHARED((N_PAD, H1), jnp.float32),
            pltpu.SemaphoreType.DMA,
            pltpu.SemaphoreType.DMA,
            pltpu.SemaphoreType.DMA,
        ],
    )(_sc_body)

    agg2 = sc(srcp, dstp, ewp, h)

    out = pl.pallas_call(
        _epi_body,
        out_shape=jax.ShapeDtypeStruct((1, W2.shape[0]), jnp.float32),
    )(agg2[:, :N], b_gcn.reshape(1, H1), gamma.reshape(1, H1),
      beta.reshape(1, H1), W1, b1.reshape(1, -1), W2, b2.reshape(1, -1))
    return out


# speedup vs baseline: 19.8296x; 1.0712x over previous
import functools

import jax
import jax.numpy as jnp
from jax import lax
from jax.experimental import pallas as pl
from jax.experimental.pallas import tpu as pltpu
from jax.experimental.pallas import tpu_sc as plsc

N = 10000
E = 320000
D_IN = 128
H1 = 64

NC = 2
NS = 16
NW = NC * NS

C = 128
EW = 10240
NCHUNK = EW // C
E_PAD = NW * EW
N_PAD = 10240
NSL = N_PAD // NS


def _mm_body(x_ref, w_ref, o_ref):
    o_ref[...] = lax.dot_general(
        x_ref[...], w_ref[...], (((1,), (1,)), ((), ())),
        preferred_element_type=jnp.float32)


def _epi_body(agg_ref, b_ref, g_ref, be_ref, w1_ref, b1_ref, w2_ref, b2_ref,
              o_ref):
    agg = agg_ref[0] + agg_ref[1]
    out = jnp.maximum(agg + b_ref[...], 0.0)
    mu = jnp.mean(out, axis=-1, keepdims=True)
    var = jnp.mean((out - mu) * (out - mu), axis=-1, keepdims=True)
    ln = (out - mu) * lax.rsqrt(var + 1e-5) * g_ref[...] + be_ref[...]
    pooled = jnp.mean(ln, axis=0, keepdims=True)
    z = lax.dot_general(pooled, w1_ref[...], (((1,), (1,)), ((), ())),
                        preferred_element_type=jnp.float32) + b1_ref[...]
    z = jnp.maximum(z, 0.0)
    o_ref[...] = lax.dot_general(z, w2_ref[...], (((1,), (1,)), ((), ())),
                                 preferred_element_type=jnp.float32) + b2_ref[...]


def _sc_body(srcp, dstp, ewp, h_hbm, agg_out,
             src_v, dst_v, ew_v, deg_v, dinv_sl, rows_v,
             norm_c, degs_s, dinv_s, agg_s, sem_g0, sem_g1, sem_s0, sem_s1,
             sem_m):
    c = lax.axis_index("c")
    s = lax.axis_index("s")
    w = s * NC + c
    t = s

    zero16 = jnp.zeros((16,), jnp.float32)

    def zbody(i, _):
        deg_v[pl.ds(i * 16, 16)] = zero16
        return ()
    lax.fori_loop(0, N_PAD // 16, zbody, ())

    for k in range(2):
        wk = t * 2 + k
        pltpu.sync_copy(dstp.at[wk], dst_v)
        pltpu.sync_copy(ewp.at[wk], ew_v)

        def dchunk(ci, _):
            for j in range(C // 16):
                sl = pl.ds(j * 16, 16)
                plsc.addupdate_scatter(deg_v, [dst_v[ci, sl]], ew_v[ci, sl])
            return ()
        lax.fori_loop(0, NCHUNK, dchunk, ())

    pltpu.sync_copy(deg_v, degs_s.at[t])
    plsc.subcore_barrier()

    descs = []
    for p in range(NS):
        descs.append(pltpu.async_copy(
            degs_s.at[p, pl.ds(t * NSL, NSL)], deg_v.at[pl.ds(p * NSL, NSL)],
            sem_m))
    for d in descs:
        d.wait()

    def nbody(i, _):
        sl = pl.ds(i * 16, 16)
        acc = deg_v[pl.ds(i * 16, 16)]
        for p in range(1, NS):
            acc = acc + deg_v[pl.ds(p * NSL + i * 16, 16)]
        dg = acc + 1.0
        ii = plsc.bitcast(dg, jnp.int32)
        ii = jnp.int32(0x5F3759DF) - lax.shift_right_arithmetic(ii, 1)
        y = plsc.bitcast(ii, jnp.float32)
        for _ in range(4):
            y = y * (1.5 - 0.5 * dg * y * y)
        dinv_sl[sl] = y
        return ()
    lax.fori_loop(0, NSL // 16, nbody, ())

    pltpu.sync_copy(dinv_sl, dinv_s.at[pl.ds(t * NSL, NSL)])
    plsc.subcore_barrier()
    pltpu.sync_copy(dinv_s, deg_v)

    for q in range(NSL // C):
        base = t * NSL + q * C
        pltpu.sync_copy(h_hbm.at[pl.ds(base, C)], rows_v.at[0])

        def sbody(g, _):
            dv = deg_v[pl.ds(base + g * 16, 16)]
            dv2 = dv * dv * 0.5
            for k in range(16):
                bv = jnp.full((16,), dv2[k], jnp.float32)
                e = g * 16 + k
                for j in range(H1 // 16):
                    sl = pl.ds(j * 16, 16)
                    rows_v[0, e, sl] = rows_v[0, e, sl] * bv
            return ()
        lax.fori_loop(0, C // 16, sbody, ())
        pltpu.sync_copy(rows_v.at[0], agg_s.at[pl.ds(base, C)])

    pltpu.sync_copy(srcp.at[w], src_v)
    pltpu.sync_copy(dstp.at[w], dst_v)
    pltpu.sync_copy(ewp.at[w], ew_v)

    def nrm(ci, _):
        for j in range(C // 16):
            sl = pl.ds(j * 16, 16)
            gs = plsc.load_gather(deg_v, [src_v[ci, sl]])
            gd = plsc.load_gather(deg_v, [dst_v[ci, sl]])
            ew_v[ci, sl] = gs * ew_v[ci, sl] * gd
        return ()
    lax.fori_loop(0, NCHUNK, nrm, ())

    plsc.subcore_barrier()

    pltpu.async_copy(h_hbm.at[src_v.at[0]], rows_v.at[0], sem_g0)
    pltpu.async_copy(h_hbm.at[src_v.at[1]], rows_v.at[1], sem_g1)

    def mainloop(step, _):
        for b in range(2):
            ci = step * 2 + b
            sem = sem_g0 if b == 0 else sem_g1
            sem_s = sem_s0 if b == 0 else sem_s1
            pltpu.make_async_copy(
                h_hbm.at[src_v.at[ci]], rows_v.at[b], sem).wait()

            @pl.when(ci >= 2)
            def _():
                pltpu.make_async_copy(
                    rows_v.at[2 + b], agg_s.at[dst_v.at[ci - 2]], sem_s).wait()

            for g in range(C // 16):
                norm_c[pl.ds(g * 16, 16)] = ew_v[ci, pl.ds(g * 16, 16)]

            def scale(g, _):
                nv = norm_c[pl.ds(g * 16, 16)]
                for k in range(16):
                    bv = jnp.full((16,), nv[k], jnp.float32)
                    e = g * 16 + k
                    for j in range(H1 // 16):
                        sl = pl.ds(j * 16, 16)
                        rows_v[2 + b, e, sl] = rows_v[b, e, sl] * bv
                return ()
            lax.fori_loop(0, C // 16, scale, ())

            @pl.when(ci + 2 < NCHUNK)
            def _():
                pltpu.async_copy(h_hbm.at[src_v.at[ci + 2]], rows_v.at[b], sem)
            pltpu.async_copy(
                rows_v.at[2 + b], agg_s.at[dst_v.at[ci]], sem_s, add=True)
        return ()
    lax.fori_loop(0, NCHUNK // 2, mainloop, ())

    pltpu.make_async_copy(
        rows_v.at[2], agg_s.at[dst_v.at[NCHUNK - 2]], sem_s0).wait()
    pltpu.make_async_copy(
        rows_v.at[3], agg_s.at[dst_v.at[NCHUNK - 1]], sem_s1).wait()

    plsc.subcore_barrier()
    pltpu.sync_copy(agg_s.at[pl.ds(t * NSL, NSL)],
                    agg_out.at[c, pl.ds(t * NSL, NSL)])


def kernel(x, edge_index, edge_weight, W_gcn, b_gcn, gamma, beta, W1, b1, W2,
           b2):
    src = edge_index[0]
    dst = edge_index[1]
    pad = E_PAD - E
    srcp = jnp.concatenate([src, jnp.zeros((pad,), src.dtype)])
    dstp = jnp.concatenate([dst, jnp.zeros((pad,), dst.dtype)])
    ewp = jnp.concatenate([edge_weight, jnp.zeros((pad,), edge_weight.dtype)])
    srcp = srcp.reshape(NW, NCHUNK, C)
    dstp = dstp.reshape(NW, NCHUNK, C)
    ewp = ewp.reshape(NW, NCHUNK, C)

    x_pad = jnp.concatenate(
        [x, jnp.zeros((N_PAD - N, D_IN), x.dtype)], axis=0)

    h = pl.pallas_call(
        _mm_body,
        out_shape=jax.ShapeDtypeStruct((N_PAD, H1), jnp.float32),
    )(x_pad, W_gcn)

    mesh = plsc.VectorSubcoreMesh(
        core_axis_name="c", subcore_axis_name="s",
        num_cores=NC, num_subcores=NS)

    sc = functools.partial(
        pl.kernel,
        out_type=jax.ShapeDtypeStruct((NC, N_PAD, H1), jnp.float32),
        mesh=mesh,
        compiler_params=pltpu.CompilerParams(
            needs_layout_passes=False, use_tc_tiling_on_sc=False),
        scratch_types=[
            pltpu.VMEM((NCHUNK, C), jnp.int32),
            pltpu.VMEM((NCHUNK, C), jnp.int32),
            pltpu.VMEM((NCHUNK, C), jnp.float32),
            pltpu.VMEM((N_PAD,), jnp.float32),
            pltpu.VMEM((NSL,), jnp.float32),
            pltpu.VMEM((4, C, H1), jnp.float32),
            pltpu.VMEM((C,), jnp.float32),
            pltpu.VMEM_SHARED((NS, N_PAD), jnp.float32),
            pltpu.VMEM_SHARED((N_PAD,), jnp.float32),
            pltpu.VMEM_SHARED((N_PAD, H1), jnp.float32),
            pltpu.SemaphoreType.DMA,
            pltpu.SemaphoreType.DMA,
            pltpu.SemaphoreType.DMA,
            pltpu.SemaphoreType.DMA,
            pltpu.SemaphoreType.DMA,
        ],
    )(_sc_body)

    agg2 = sc(srcp, dstp, ewp, h)

    out = pl.pallas_call(
        _epi_body,
        out_shape=jax.ShapeDtypeStruct((1, W2.shape[0]), jnp.float32),
    )(agg2[:, :N], b_gcn.reshape(1, H1), gamma.reshape(1, H1),
      beta.reshape(1, H1), W1, b1.reshape(1, -1), W2, b2.reshape(1, -1))
    return out
